# Initial kernel scaffold; baseline (speedup 1.0000x reference)
import functools

import jax
import jax.numpy as jnp
from jax import lax
from jax.experimental import pallas as pl
from jax.experimental.pallas import tpu as pltpu
from jax.experimental.pallas import tpu_sc as plsc

_D = 32
_NW = 32
_CHUNK = 128
_K = 8


def _emb_body(x_hbm, table_hbm, out_hbm, idx_v, rows_v, sem_g, sem_s,
              *, nch, per_w):
    wid = lax.axis_index("s") * 2 + lax.axis_index("c")
    base = wid * per_w
    pltpu.sync_copy(x_hbm.at[wid], idx_v)

    def gather_desc(j, slot):
        return pltpu.make_async_copy(
            table_hbm.at[idx_v.at[j]], rows_v.at[slot], sem_g)

    def store_desc(j, slot):
        return pltpu.make_async_copy(
            rows_v.at[slot], out_hbm.at[pl.ds(base + j * _CHUNK, _CHUNK)],
            sem_s)

    def outer(g, _):
        for b in range(_K):
            gather_desc(g * _K + b, b).start()
        for b in range(_K):
            gather_desc(g * _K + b, b).wait()
        for b in range(_K):
            store_desc(g * _K + b, b).start()
        for b in range(_K):
            store_desc(g * _K + b, b).wait()
        return 0

    lax.fori_loop(0, nch // _K, outer, 0)


def kernel(x, table):
    B, L = x.shape
    total = B * L
    per_w = total // _NW
    nch = per_w // _CHUNK
    assert per_w % _CHUNK == 0 and nch % _K == 0

    mesh = plsc.VectorSubcoreMesh(core_axis_name="c", subcore_axis_name="s")
    body = functools.partial(_emb_body, nch=nch, per_w=per_w)
    emb = pl.kernel(
        body,
        out_type=jax.ShapeDtypeStruct((total, _D), jnp.float32),
        mesh=mesh,
        scratch_types=[
            pltpu.VMEM((nch, _CHUNK), jnp.int32),
            pltpu.VMEM((_K, _CHUNK, _D), jnp.float32),
            pltpu.SemaphoreType.DMA,
            pltpu.SemaphoreType.DMA,
        ],
    )
    x_flat = x.reshape(_NW, nch, _CHUNK)
    out = emb(x_flat, table)
    return out.reshape(B, L, _D)


# speedup vs baseline: 1.1029x; 1.1029x over previous
import functools

import jax
import jax.numpy as jnp
from jax import lax
from jax.experimental import pallas as pl
from jax.experimental.pallas import tpu as pltpu
from jax.experimental.pallas import tpu_sc as plsc

_D = 32
_NW = 32
_CHUNK = 128
_K = 8


def _emb_body(x_hbm, table_hbm, out_hbm, idx_v, rows_v, sem_g, sem_s,
              *, nch, per_w):
    wid = lax.axis_index("s") * 2 + lax.axis_index("c")
    base = wid * per_w
    pltpu.sync_copy(x_hbm.at[wid], idx_v)

    def gather_desc(j, slot):
        return pltpu.make_async_copy(
            table_hbm.at[idx_v.at[j]], rows_v.at[slot], sem_g)

    def store_desc(j, slot):
        return pltpu.make_async_copy(
            rows_v.at[slot], out_hbm.at[pl.ds(base + j * _CHUNK, _CHUNK)],
            sem_s)

    def outer(g, _):
        for b in range(_K):
            gather_desc(g * _K + b, b).start()
        for b in range(_K):
            gather_desc(g * _K + b, b).wait()
        for b in range(_K):
            store_desc(g * _K + b, b).start()
        for b in range(_K):
            store_desc(g * _K + b, b).wait()
        return 0

    lax.fori_loop(0, nch // _K, outer, 0)


def kernel(x, table):
    B, L = x.shape
    total = B * L
    per_w = total // _NW
    nch = per_w // _CHUNK
    assert per_w % _CHUNK == 0 and nch % _K == 0

    mesh = plsc.VectorSubcoreMesh(core_axis_name="c", subcore_axis_name="s")
    body = functools.partial(_emb_body, nch=nch, per_w=per_w)
    emb = pl.kernel(
        body,
        out_type=jax.ShapeDtypeStruct((total, _D), jnp.float32),
        mesh=mesh,
        scratch_types=[
            pltpu.VMEM((nch, _CHUNK), jnp.int32),
            pltpu.VMEM((_K, _CHUNK, _D), jnp.float32),
            pltpu.SemaphoreType.DMA,
            pltpu.SemaphoreType.DMA,
        ],
        compiler_params=pltpu.CompilerParams(use_tc_tiling_on_sc=False),
    )
    x_flat = x.reshape(_NW, nch, _CHUNK)
    out = emb(x_flat, table)
    return out.reshape(B, L, _D)


# speedup vs baseline: 1.3065x; 1.1846x over previous
import functools

import jax
import jax.numpy as jnp
from jax import lax
from jax.experimental import pallas as pl
from jax.experimental.pallas import tpu as pltpu
from jax.experimental.pallas import tpu_sc as plsc

_D = 32
_NW = 32
_CHUNK = 128
_K = 10


def _emb_body(x_hbm, table_hbm, out_hbm, idx_v, rows_a, rows_b,
              sem_g, sem_sa, sem_sb, *, ngroups, nch):
    wid = lax.axis_index("s") * 2 + lax.axis_index("c")
    chunk_base = wid * nch
    pltpu.sync_copy(x_hbm.at[wid], idx_v)

    def gather(g, b, buf):
        return pltpu.make_async_copy(
            table_hbm.at[idx_v.at[g * _K + b]], buf.at[b], sem_g)

    def fire_gathers(g, buf):
        for b in range(_K):
            gather(g, b, buf).start()

    def drain_gathers(g, buf):
        for b in range(_K):
            gather(g, b, buf).wait()

    def store(g, buf, sem):
        return pltpu.make_async_copy(
            buf, out_hbm.at[pl.ds(chunk_base + g * _K, _K)], sem)

    fire_gathers(0, rows_a)

    def outer(h, _):
        ga = 2 * h
        gb = 2 * h + 1
        drain_gathers(ga, rows_a)

        @pl.when(h > 0)
        def _():
            store(gb - 2, rows_b, sem_sb).wait()
        fire_gathers(gb, rows_b)
        store(ga, rows_a, sem_sa).start()
        drain_gathers(gb, rows_b)
        store(ga, rows_a, sem_sa).wait()

        @pl.when(h < ngroups // 2 - 1)
        def _():
            fire_gathers(ga + 2, rows_a)
        store(gb, rows_b, sem_sb).start()
        return 0

    lax.fori_loop(0, ngroups // 2, outer, 0)
    store(ngroups - 1, rows_b, sem_sb).wait()


def kernel(x, table):
    B, L = x.shape
    total = B * L
    per_w = total // _NW
    nch = per_w // _CHUNK
    ngroups = nch // _K
    assert per_w % _CHUNK == 0 and nch % _K == 0 and ngroups % 2 == 0

    mesh = plsc.VectorSubcoreMesh(core_axis_name="c", subcore_axis_name="s")
    body = functools.partial(_emb_body, ngroups=ngroups, nch=nch)
    rows_t = pltpu.VMEM((_K, _CHUNK, _D), jnp.float32)
    emb = pl.kernel(
        body,
        out_type=jax.ShapeDtypeStruct((total // _CHUNK, _CHUNK, _D),
                                      jnp.float32),
        mesh=mesh,
        scratch_types=[
            pltpu.VMEM((nch, _CHUNK), jnp.int32),
            rows_t,
            rows_t,
            pltpu.SemaphoreType.DMA,
            pltpu.SemaphoreType.DMA,
            pltpu.SemaphoreType.DMA,
        ],
        compiler_params=pltpu.CompilerParams(use_tc_tiling_on_sc=False),
    )
    x_flat = x.reshape(_NW, nch, _CHUNK)
    out = emb(x_flat, table)
    return out.reshape(B, L, _D)


# speedup vs baseline: 1.7921x; 1.3717x over previous
import functools

import jax
import jax.numpy as jnp
from jax import lax
from jax.experimental import pallas as pl
from jax.experimental.pallas import tpu as pltpu
from jax.experimental.pallas import tpu_sc as plsc

_D = 32
_NW = 32
_K = 16


def _emb_body(x_hbm, table_hbm, out_hbm, idx_v, rows_a, rows_b,
              sem_g, sem_sa, sem_sb, *, L, ngroups, rows_per_w):
    wid = lax.axis_index("s") * 2 + lax.axis_index("c")
    row_base = wid * rows_per_w
    pltpu.sync_copy(x_hbm.at[wid], idx_v)

    def gather(g, b, buf):
        return pltpu.make_async_copy(
            table_hbm.at[idx_v.at[g * _K + b]], buf.at[b], sem_g)

    def fire_gathers(g, buf):
        for b in range(_K):
            gather(g, b, buf).start()

    def drain_gathers(g, buf):
        for b in range(_K):
            gather(g, b, buf).wait()

    def store(g, buf, sem):
        return pltpu.make_async_copy(
            buf, out_hbm.at[pl.ds(row_base + g * _K, _K)], sem)

    fire_gathers(0, rows_a)

    def outer(h, _):
        ga = 2 * h
        gb = 2 * h + 1
        drain_gathers(ga, rows_a)

        @pl.when(h > 0)
        def _():
            store(gb - 2, rows_b, sem_sb).wait()
        fire_gathers(gb, rows_b)
        store(ga, rows_a, sem_sa).start()
        drain_gathers(gb, rows_b)
        store(ga, rows_a, sem_sa).wait()

        @pl.when(h < ngroups // 2 - 1)
        def _():
            fire_gathers(ga + 2, rows_a)
        store(gb, rows_b, sem_sb).start()
        return 0

    lax.fori_loop(0, ngroups // 2, outer, 0)
    store(ngroups - 1, rows_b, sem_sb).wait()


def kernel(x, table):
    B, L = x.shape
    rows_per_w = B // _NW
    ngroups = rows_per_w // _K
    assert B % _NW == 0 and rows_per_w % _K == 0 and ngroups % 2 == 0

    mesh = plsc.VectorSubcoreMesh(core_axis_name="c", subcore_axis_name="s")
    body = functools.partial(_emb_body, L=L, ngroups=ngroups,
                             rows_per_w=rows_per_w)
    rows_t = pltpu.VMEM((_K, L, _D), jnp.float32)
    emb = pl.kernel(
        body,
        out_type=jax.ShapeDtypeStruct((B, L, _D), jnp.float32),
        mesh=mesh,
        scratch_types=[
            pltpu.VMEM((rows_per_w, L), jnp.int32),
            rows_t,
            rows_t,
            pltpu.SemaphoreType.DMA,
            pltpu.SemaphoreType.DMA,
            pltpu.SemaphoreType.DMA,
        ],
        compiler_params=pltpu.CompilerParams(use_tc_tiling_on_sc=False),
    )
    x_split = x.reshape(_NW, rows_per_w, L)
    return emb(x_split, table)
